# baseline (speedup 1.0000x reference)
import functools

import numpy as np
import jax
import jax.numpy as jnp
from jax import lax
from jax.experimental import pallas as pl
from jax.experimental.pallas import tpu as pltpu
from jax.experimental.pallas import tpu_sc as plsc

_NCLS = 64
_T = 100
_TPAD = 128
_LOG_NC = float(np.log(_NCLS))
_LOG_EPS = float(np.log(np.float32(1e-30)))


def _block_body(ts_ref, la_ref, l1ma_ref, pk_ref, u_ref, vp_ref, lnvt_ref):
    f32 = jnp.float32
    log_eps = jnp.log(f32(1e-30))

    def lae(a, b):
        m = jnp.maximum(a, b)
        return m + jnp.log(jnp.exp(a - m) + jnp.exp(b - m))

    la = la_ref[...]
    l1ma = l1ma_ref[...]
    rest = l1ma - _LOG_NC
    on_col = lae(la, rest)
    off_col = lae(la + log_eps, rest)

    ts = ts_ref[...]
    iota_sub = lax.broadcasted_iota(jnp.int32, (_TPAD, _NCLS), 0)
    mt = ts == iota_sub
    on_b = jnp.sum(jnp.where(mt, on_col, f32(0.0)), axis=0, keepdims=True)
    off_b = jnp.sum(jnp.where(mt, off_col, f32(0.0)), axis=0, keepdims=True)

    pk = pk_ref[...]
    bidx = lax.shift_right_logical(pk, 6)
    vcls = lax.bitwise_and(pk, _NCLS - 1)
    iota64 = lax.broadcasted_iota(jnp.int32, (1, _NCLS), 1)
    mb = bidx == iota64
    on_n = jnp.sum(jnp.where(mb, on_b, f32(0.0)), axis=1, keepdims=True)
    off_n = jnp.sum(jnp.where(mb, off_b, f32(0.0)), axis=1, keepdims=True)

    u = u_ref[...]
    g = -jnp.log(-jnp.log(u + f32(1e-30)) + f32(1e-30))
    mv = vcls == iota64
    val = g + jnp.where(mv, on_n, off_n)
    vmax = jnp.max(val, axis=1, keepdims=True)
    samp = jnp.min(jnp.where(val == vmax, iota64, _NCLS), axis=1, keepdims=True)
    ms = samp == iota64

    vp_ref[...] = jnp.where(ms, f32(1.0), f32(0.0))
    lnvt_ref[...] = jnp.where(ms, f32(0.0), log_eps)


def _make_lv0_sc(n):
    info = plsc.get_sparse_core_info()
    nw = info.num_cores * info.num_subcores
    per_w = n // nw
    ch = 512
    chunks = per_w // ch
    groups = ch // 16
    mesh = plsc.VectorSubcoreMesh(core_axis_name="c", subcore_axis_name="s")

    @functools.partial(
        pl.kernel, mesh=mesh,
        out_type=jax.ShapeDtypeStruct((n * _NCLS,), jnp.float32),
        scratch_types=[
            pltpu.VMEM((per_w,), jnp.int32),
            pltpu.VMEM((ch * _NCLS,), jnp.float32),
            pltpu.VMEM((per_w,), jnp.float32),
            pltpu.VMEM((per_w,), jnp.int32),
            pltpu.SemaphoreType.DMA,
            pltpu.SemaphoreType.DMA,
        ],
    )
    def lv0_sc(v_hbm, out_hbm, vvm, bg, zeros_v, idxv, sem_a, sem_b):
        f32 = jnp.float32
        eps16 = jnp.full((16,), _LOG_EPS, f32)
        zero16 = jnp.zeros((16,), f32)
        iota16 = lax.iota(jnp.int32, 16)
        wid = lax.axis_index("s") * info.num_cores + lax.axis_index("c")
        base = wid * per_w
        pltpu.sync_copy(v_hbm.at[pl.ds(base, per_w)], vvm)

        def fill_bg(i, carry):
            bg[pl.ds(i * 16, 16)] = eps16
            return carry

        lax.fori_loop(0, ch * _NCLS // 16, fill_bg, 0)

        def fill_idx(i, carry):
            zeros_v[pl.ds(i * 16, 16)] = zero16
            vv = vvm[pl.ds(i * 16, 16)]
            idxv[pl.ds(i * 16, 16)] = (base + i * 16 + iota16) * _NCLS + vv
            return carry

        lax.fori_loop(0, per_w // 16, fill_idx, 0)

        handles = []
        for c in range(chunks):
            handles.append(pltpu.async_copy(
                bg, out_hbm.at[pl.ds((base + c * ch) * _NCLS, ch * _NCLS)],
                sem_a))
        for h in handles:
            h.wait()
        pltpu.async_copy(zeros_v, out_hbm.at[idxv], sem_b).wait()

    return lv0_sc


def kernel(v, time_step, batch, u, log_alphas_bar, log_1_min_alphas_bar):
    n = u.shape[0]
    rows = 8192
    grid = n // rows
    ts2 = time_step.reshape(1, _NCLS)
    la2 = jnp.pad(log_alphas_bar, (0, _TPAD - _T)).reshape(_TPAD, 1)
    l12 = jnp.pad(log_1_min_alphas_bar, (0, _TPAD - _T)).reshape(_TPAD, 1)
    pk = (batch * _NCLS + v).reshape(n, 1)

    lv0 = _make_lv0_sc(n)(v).reshape(n, _NCLS)

    grid_spec = pl.GridSpec(
        grid=(grid,),
        in_specs=[
            pl.BlockSpec((1, _NCLS), lambda i: (0, 0)),
            pl.BlockSpec((_TPAD, 1), lambda i: (0, 0)),
            pl.BlockSpec((_TPAD, 1), lambda i: (0, 0)),
            pl.BlockSpec((rows, 1), lambda i: (i, 0)),
            pl.BlockSpec((rows, _NCLS), lambda i: (i, 0)),
        ],
        out_specs=[pl.BlockSpec((rows, _NCLS), lambda i: (i, 0))] * 2,
    )
    vp, lnvt = pl.pallas_call(
        _block_body,
        grid_spec=grid_spec,
        out_shape=[jax.ShapeDtypeStruct((n, _NCLS), jnp.float32)] * 2,
        compiler_params=pltpu.CompilerParams(
            dimension_semantics=("parallel",)),
    )(ts2, la2, l12, pk, u)
    return (vp, lnvt, lv0)


# speedup vs baseline: 1.6607x; 1.6607x over previous
import numpy as np
import jax
import jax.numpy as jnp
from jax.experimental import pallas as pl
from jax.experimental.pallas import tpu as pltpu

_NCLS = 64
_T = 100
_TPAD = 128
_LOG_NC = float(np.log(_NCLS))


def _block_body(ts_ref, la_ref, l1ma_ref, pk_ref, u_ref,
                vp_ref, lnvt_ref, lv0_ref):
    f32 = jnp.float32
    log_eps = jnp.log(f32(1e-30))

    def lae(a, b):
        m = jnp.maximum(a, b)
        return m + jnp.log(jnp.exp(a - m) + jnp.exp(b - m))

    la = la_ref[...]
    l1ma = l1ma_ref[...]
    rest = l1ma - _LOG_NC
    on_col = lae(la, rest)
    off_col = lae(la + log_eps, rest)

    ts = ts_ref[...]
    iota_sub = jax.lax.broadcasted_iota(jnp.int32, (_TPAD, _NCLS), 0)
    mt = ts == iota_sub
    on_b = jnp.sum(jnp.where(mt, on_col, f32(0.0)), axis=0, keepdims=True)
    off_b = jnp.sum(jnp.where(mt, off_col, f32(0.0)), axis=0, keepdims=True)

    pk = pk_ref[...]
    bidx = jax.lax.shift_right_logical(pk, 6)
    vcls = jax.lax.bitwise_and(pk, _NCLS - 1)
    iota64 = jax.lax.broadcasted_iota(jnp.int32, (1, _NCLS), 1)
    mb = bidx == iota64
    on_n = jnp.sum(jnp.where(mb, on_b, f32(0.0)), axis=1, keepdims=True)
    off_n = jnp.sum(jnp.where(mb, off_b, f32(0.0)), axis=1, keepdims=True)

    u = u_ref[...]
    g = -jnp.log(-jnp.log(u + f32(1e-30)) + f32(1e-30))
    mv = vcls == iota64
    val = g + jnp.where(mv, on_n, off_n)
    vmax = jnp.max(val, axis=1, keepdims=True)
    ms = val == vmax

    vp_ref[...] = jnp.where(ms, f32(1.0), f32(0.0))
    lnvt_ref[...] = jnp.where(ms, f32(0.0), log_eps)
    lv0_ref[...] = jnp.where(mv, f32(0.0), log_eps)


def kernel(v, time_step, batch, u, log_alphas_bar, log_1_min_alphas_bar):
    n = u.shape[0]
    rows = 8192
    grid = n // rows
    ts2 = time_step.reshape(1, _NCLS)
    la2 = jnp.pad(log_alphas_bar, (0, _TPAD - _T)).reshape(_TPAD, 1)
    l12 = jnp.pad(log_1_min_alphas_bar, (0, _TPAD - _T)).reshape(_TPAD, 1)
    pk = (batch * _NCLS + v).reshape(n, 1)

    grid_spec = pl.GridSpec(
        grid=(grid,),
        in_specs=[
            pl.BlockSpec((1, _NCLS), lambda i: (0, 0)),
            pl.BlockSpec((_TPAD, 1), lambda i: (0, 0)),
            pl.BlockSpec((_TPAD, 1), lambda i: (0, 0)),
            pl.BlockSpec((rows, 1), lambda i: (i, 0)),
            pl.BlockSpec((rows, _NCLS), lambda i: (i, 0)),
        ],
        out_specs=[pl.BlockSpec((rows, _NCLS), lambda i: (i, 0))] * 3,
    )
    vp, lnvt, lv0 = pl.pallas_call(
        _block_body,
        grid_spec=grid_spec,
        out_shape=[jax.ShapeDtypeStruct((n, _NCLS), jnp.float32)] * 3,
        compiler_params=pltpu.CompilerParams(
            dimension_semantics=("parallel",)),
    )(ts2, la2, l12, pk, u)
    return (vp, lnvt, lv0)


# speedup vs baseline: 1.7380x; 1.0465x over previous
import numpy as np
import jax
import jax.numpy as jnp
from jax.experimental import pallas as pl
from jax.experimental.pallas import tpu as pltpu

_NCLS = 64
_T = 100
_TPAD = 128
_LOG_NC = float(np.log(_NCLS))


def _block_body(ts_ref, la_ref, l1ma_ref, pk_ref, u_ref,
                vp_ref, lnvt_ref, lv0_ref):
    f32 = jnp.float32
    log_eps = jnp.log(f32(1e-30))

    def lae(a, b):
        m = jnp.maximum(a, b)
        return m + jnp.log(jnp.exp(a - m) + jnp.exp(b - m))

    la = la_ref[...]
    l1ma = l1ma_ref[...]
    rest = l1ma - _LOG_NC
    on_col = lae(la, rest)
    off_col = lae(la + log_eps, rest)

    ts = ts_ref[...]
    iota_sub = jax.lax.broadcasted_iota(jnp.int32, (_TPAD, _NCLS), 0)
    mt = ts == iota_sub
    on_b = jnp.sum(jnp.where(mt, on_col, f32(0.0)), axis=0, keepdims=True)
    off_b = jnp.sum(jnp.where(mt, off_col, f32(0.0)), axis=0, keepdims=True)

    pk = pk_ref[...].astype(jnp.int32)
    bidx = jax.lax.shift_right_logical(pk, 6)
    vcls = jax.lax.bitwise_and(pk, _NCLS - 1)
    iota64 = jax.lax.broadcasted_iota(jnp.int32, (1, _NCLS), 1)
    mb = bidx == iota64
    on_n = jnp.sum(jnp.where(mb, on_b, f32(0.0)), axis=1, keepdims=True)
    off_n = jnp.sum(jnp.where(mb, off_b, f32(0.0)), axis=1, keepdims=True)

    u = u_ref[...]
    g = -jnp.log(-jnp.log(u + f32(1e-30)) + f32(1e-30))
    mv = vcls == iota64
    val = g + jnp.where(mv, on_n, off_n)
    vmax = jnp.max(val, axis=1, keepdims=True)
    ms = val == vmax

    vp_ref[...] = jnp.where(ms, f32(1.0), f32(0.0))
    lnvt_ref[...] = jnp.where(ms, f32(0.0), log_eps)
    lv0_ref[...] = jnp.where(mv, f32(0.0), log_eps)


def kernel(v, time_step, batch, u, log_alphas_bar, log_1_min_alphas_bar):
    n = u.shape[0]
    rows = 8192
    grid = n // rows
    ts2 = time_step.reshape(1, _NCLS)
    la2 = jnp.pad(log_alphas_bar, (0, _TPAD - _T)).reshape(_TPAD, 1)
    l12 = jnp.pad(log_1_min_alphas_bar, (0, _TPAD - _T)).reshape(_TPAD, 1)
    pk = (batch * _NCLS + v).astype(jnp.int16).reshape(n, 1)

    grid_spec = pl.GridSpec(
        grid=(grid,),
        in_specs=[
            pl.BlockSpec((1, _NCLS), lambda i: (0, 0)),
            pl.BlockSpec((_TPAD, 1), lambda i: (0, 0)),
            pl.BlockSpec((_TPAD, 1), lambda i: (0, 0)),
            pl.BlockSpec((rows, 1), lambda i: (i, 0)),
            pl.BlockSpec((rows, _NCLS), lambda i: (i, 0)),
        ],
        out_specs=[pl.BlockSpec((rows, _NCLS), lambda i: (i, 0))] * 3,
    )
    vp, lnvt, lv0 = pl.pallas_call(
        _block_body,
        grid_spec=grid_spec,
        out_shape=[jax.ShapeDtypeStruct((n, _NCLS), jnp.float32)] * 3,
        compiler_params=pltpu.CompilerParams(
            dimension_semantics=("parallel",)),
    )(ts2, la2, l12, pk, u)
    return (vp, lnvt, lv0)
